# Initial kernel scaffold; baseline (speedup 1.0000x reference)
import functools

import jax
import jax.numpy as jnp
import numpy as np
from jax.experimental import pallas as pl
from jax.experimental.pallas import tpu as pltpu

A, NN, F, RES, NI = 128, 48, 128, 20, 3
E = A * NN
CUTOFF = 5.0
P = 9.0
EPS = 1e-8


def _mm(x, w):
    return jax.lax.dot_general(x, w, (((1,), (1,)), ((), ())),
                               preferred_element_type=jnp.float32)


def _mmT(x, w):
    return jax.lax.dot_general(x, w, (((1,), (0,)), ((), ())),
                               preferred_element_type=jnp.float32)


def _swish(z):
    return z * jax.nn.sigmoid(z)


def _swish_d(z):
    s = jax.nn.sigmoid(z)
    return s * (1.0 + z * (1.0 - s))


def _rep(x):
    f = x.shape[-1]
    return jnp.broadcast_to(x[:, None, :], (A, NN, f)).reshape(E, f)


def _seg(x):
    f = x.shape[-1]
    return jnp.sum(x.reshape(A, NN, f), axis=1)


def _mul_nm(x, nm):
    f = x.shape[-1]
    return (x.reshape(A, NN, f) * nm[:, :, None]).reshape(E, f)


def _dense_fwd(p, x):
    z = _mm(x, p['W'])
    if 'b' in p:
        z = z + p['b']
    return z


def _mlp2(p0, p1, x):
    z0 = _dense_fwd(p0, x)
    return z0, _dense_fwd(p1, _swish(z0))


def _mlp2_bwd(p0, p1, z0, g_out):
    g_h = _mmT(g_out, p1['W'])
    return _mmT(g_h * _swish_d(z0), p0['W'])


def _newton_kernel(treedef, r_ref, z_ref, n_ref, am_ref, nm_ref, *refs):
    param_refs = jax.tree_util.tree_unflatten(treedef, refs[:-3])
    prm = jax.tree_util.tree_map(lambda r: r[...], param_refs)
    e_ref, ff_ref, fdir_ref = refs[-3:]

    Rm = r_ref[0]
    Zc = z_ref[0]
    Nm = n_ref[0]
    AMc = am_ref[0]
    NMm = nm_ref[0]

    layers = prm['layers']
    emb = prm['emb']
    atm = prm['atomic']

    ids = jax.lax.broadcasted_iota(jnp.int32, (A, NN, A), 2)
    O = (Nm[:, :, None] == ids).astype(jnp.float32).reshape(E, A)

    def gat(x):
        return jax.lax.dot_general(O, x, (((1,), (0,)), ((), ())),
                                   preferred_element_type=jnp.float32)

    def scat(y):
        return jax.lax.dot_general(O, y, (((0,), (0,)), ((), ())),
                                   preferred_element_type=jnp.float32)

    R_c = [Rm[:, c:c + 1] for c in range(3)]
    vec = [gat(rc) - _rep(rc) for rc in R_c]
    D2 = vec[0] * vec[0] + vec[1] * vec[1] + vec[2] * vec[2] + EPS
    D = jnp.sqrt(D2)
    Dp = D + EPS
    V = [v / Dp for v in vec]
    nv = jnp.arange(1, RES + 1, dtype=jnp.float32)[None, :]
    w = nv * (np.pi / CUTOFF)
    c0 = np.sqrt(2.0 / CUTOFF)
    sin_wd = jnp.sin(w * D)
    rbf = c0 * sin_wd / Dp
    x = D / CUTOFF
    x8 = (x * x) * (x * x)
    x8 = x8 * x8
    x9 = x8 * x
    x10 = x9 * x
    x11 = x10 * x
    c_a = (P + 1.0) * (P + 2.0) / 2.0
    c_b = P * (P + 2.0)
    c_c = P * (P + 1.0) / 2.0
    in_r = x < 1.0
    cut = jnp.where(in_r, 1.0 - c_a * x9 + c_b * x10 - c_c * x11, 0.0)

    zid = jax.lax.broadcasted_iota(jnp.int32, (A, 10), 1)
    a = jax.lax.dot_general((Zc == zid).astype(jnp.float32), emb,
                            (((1,), (0,)), ((), ())),
                            preferred_element_type=jnp.float32)

    zeros_af = jnp.zeros((A, F), jnp.float32)
    r_dyn = [zeros_af] * 3
    f_dyn = [zeros_af] * 3
    fdir = [jnp.zeros((A, 1), jnp.float32)] * 3

    a_sv, r_sv, f_sv = [a], [r_dyn], [f_dyn]

    def edge_tensors(lp, a_in):
        rbf_lin = _dense_fwd(lp['phi_rbf'], rbf)
        rbf_m = rbf_lin * cut
        za0, a_m = _mlp2(lp['phi_a'][0], lp['phi_a'][1], a_in)
        a_rep = _rep(a_m)
        ag = gat(a_m)
        msij = a_rep * ag * rbf_m
        s = _mm(msij, lp['phi_f']['W'])
        Fij = [s * vc for vc in V]
        zfs0, fs = _mlp2(lp['phi_f_scale'][0], lp['phi_f_scale'][1], msij)
        Fi = [_seg(_mul_nm(fs * Fij[c], NMm)) for c in range(3)]
        zre0, pre = _mlp2(lp['phi_r_ext'][0], lp['phi_r_ext'][1], msij)
        return (rbf_lin, rbf_m, za0, a_m, a_rep, ag, msij, s, Fij,
                zfs0, fs, Fi, zre0, pre)

    for lp in layers:
        (rbf_lin, rbf_m, za0, a_m, a_rep, ag, msij, s, Fij,
         zfs0, fs, Fi, zre0, pre) = edge_tensors(lp, a)
        fdir = [fdir[c] + _seg(_mul_nm(Fij[c], NMm)) for c in range(3)]
        f_dyn = [f_dyn[c] + Fi[c] for c in range(3)]
        zr0, pr = _mlp2(lp['phi_r'][0], lp['phi_r'][1], a)
        dr_ext = [_seg(_mul_nm(pre * gat(r_dyn[c]), NMm)) for c in range(3)]
        r_dyn = [r_dyn[c] + pr * Fi[c] + dr_ext[c] for c in range(3)]
        de_raw = -(f_dyn[0] * r_dyn[0] + f_dyn[1] * r_dyn[1]
                   + f_dyn[2] * r_dyn[2])
        ze0, ee = _mlp2(lp['phi_e'][0], lp['phi_e'][1], a)
        a = a + ee * de_raw
        a_sv.append(a)
        r_sv.append(r_dyn)
        f_sv.append(f_dyn)

    zh1 = _dense_fwd(atm[0], a)
    h1 = _swish(zh1)
    zh2 = _dense_fwd(atm[1], h1)
    h2 = _swish(zh2)
    Ei = _dense_fwd(atm[2], h2) * AMc
    Etot = jnp.sum(Ei)

    g_h2 = _mmT(AMc, atm[2]['W'])
    g_h1 = _mmT(g_h2 * _swish_d(zh2), atm[1]['W'])
    g_a = _mmT(g_h1 * _swish_d(zh1), atm[0]['W'])

    g_r = [zeros_af] * 3
    g_f = [zeros_af] * 3
    g_V = [jnp.zeros((E, 1), jnp.float32)] * 3
    g_cut = jnp.zeros((E, 1), jnp.float32)
    g_rbf = jnp.zeros((E, RES), jnp.float32)

    for li in range(NI - 1, -1, -1):
        lp = layers[li]
        a_in = a_sv[li]
        r_in, f_in = r_sv[li], f_sv[li]
        r_out, f_out = r_sv[li + 1], f_sv[li + 1]

        (rbf_lin, rbf_m, za0, a_m, a_rep, ag, msij, s, Fij,
         zfs0, fs, Fi, zre0, pre) = edge_tensors(lp, a_in)
        zr0, pr = _mlp2(lp['phi_r'][0], lp['phi_r'][1], a_in)
        ze0, ee = _mlp2(lp['phi_e'][0], lp['phi_e'][1], a_in)
        de_raw = -(f_out[0] * r_out[0] + f_out[1] * r_out[1]
                   + f_out[2] * r_out[2])

        g_ee = g_a * de_raw
        g_deraw = g_a * ee
        g_a_in = g_a + _mlp2_bwd(lp['phi_e'][0], lp['phi_e'][1], ze0, g_ee)

        g_f = [g_f[c] - g_deraw * r_out[c] for c in range(3)]
        g_r = [g_r[c] - g_deraw * f_out[c] for c in range(3)]

        g_pr = g_r[0] * Fi[0] + g_r[1] * Fi[1] + g_r[2] * Fi[2]
        g_Fi = [g_r[c] * pr + g_f[c] for c in range(3)]
        g_a_in = g_a_in + _mlp2_bwd(lp['phi_r'][0], lp['phi_r'][1], zr0, g_pr)

        g_pre = jnp.zeros((E, F), jnp.float32)
        g_r_new = []
        for c in range(3):
            rg_c = gat(r_in[c])
            gdx_c = _mul_nm(_rep(g_r[c]), NMm)
            g_pre = g_pre + gdx_c * rg_c
            g_r_new.append(g_r[c] + scat(gdx_c * pre))
        g_r = g_r_new
        g_msij = _mlp2_bwd(lp['phi_r_ext'][0], lp['phi_r_ext'][1], zre0,
                           g_pre)

        g_fs = jnp.zeros((E, F), jnp.float32)
        g_s = jnp.zeros((E, 1), jnp.float32)
        for c in range(3):
            g_Fij2_c = _mul_nm(_rep(g_Fi[c]), NMm)
            g_fs = g_fs + g_Fij2_c * Fij[c]
            g_Fij_c = jnp.sum(g_Fij2_c * fs, axis=1, keepdims=True)
            g_s = g_s + g_Fij_c * V[c]
            g_V[c] = g_V[c] + g_Fij_c * s
        g_msij = g_msij + _mlp2_bwd(lp['phi_f_scale'][0],
                                    lp['phi_f_scale'][1], zfs0, g_fs)
        g_msij = g_msij + g_s * lp['phi_f']['W']

        g_arep = g_msij * ag * rbf_m
        g_ag = g_msij * a_rep * rbf_m
        g_rbfm = g_msij * a_rep * ag
        g_am = _seg(g_arep) + scat(g_ag)
        g_a_in = g_a_in + _mlp2_bwd(lp['phi_a'][0], lp['phi_a'][1], za0,
                                    g_am)

        g_cut = g_cut + jnp.sum(g_rbfm * rbf_lin, axis=1, keepdims=True)
        g_rbf = g_rbf + _mmT(g_rbfm * cut, lp['phi_rbf']['W'])

        g_a = g_a_in

    g_D = jnp.sum(g_rbf * (c0 * w * jnp.cos(w * D)) / Dp, axis=1,
                  keepdims=True)
    g_Dp = jnp.sum(g_rbf * (-c0 * sin_wd / (Dp * Dp)), axis=1,
                   keepdims=True)
    dcut = jnp.where(in_r,
                     (-9.0 * c_a) * x8 + (10.0 * c_b) * x9
                     - (11.0 * c_c) * x10, 0.0) / CUTOFF
    g_D = g_D + g_cut * dcut
    g_vec = [g_V[c] / Dp for c in range(3)]
    g_Dp = g_Dp - (g_V[0] * vec[0] + g_V[1] * vec[1]
                   + g_V[2] * vec[2]) / (Dp * Dp)
    g_D = g_D + g_Dp
    g_vec = [g_vec[c] + g_D * vec[c] / D for c in range(3)]

    fforce = [-(scat(g_vec[c]) - _seg(g_vec[c])) for c in range(3)]

    e_ref[...] = jnp.broadcast_to(Etot, (1, 1, 128))
    ff_ref[...] = jnp.concatenate(fforce, axis=1)[None]
    fdir_ref[...] = jnp.concatenate(fdir, axis=1)[None]


@functools.partial(jax.jit, static_argnames=('interpret',))
def _run(R, Z, N, AM, NM, params, interpret=False):
    B = R.shape[0]
    params2 = jax.tree_util.tree_map(
        lambda x: x.reshape((1, -1)) if x.ndim == 1 else x, dict(params))
    flat, treedef = jax.tree_util.tree_flatten(params2)

    w_specs = [pl.BlockSpec(f.shape, lambda b, sh=f.shape: (0,) * len(sh))
               for f in flat]
    out_specs = [
        pl.BlockSpec((1, 1, 128), lambda b: (b, 0, 0)),
        pl.BlockSpec((1, A, 3), lambda b: (b, 0, 0)),
        pl.BlockSpec((1, A, 3), lambda b: (b, 0, 0)),
    ]
    out_shape = [
        jax.ShapeDtypeStruct((B, 1, 128), jnp.float32),
        jax.ShapeDtypeStruct((B, A, 3), jnp.float32),
        jax.ShapeDtypeStruct((B, A, 3), jnp.float32),
    ]

    e3, ff, fdir = pl.pallas_call(
        functools.partial(_newton_kernel, treedef),
        grid=(B,),
        in_specs=[
            pl.BlockSpec((1, A, 3), lambda b: (b, 0, 0)),
            pl.BlockSpec((1, A, 1), lambda b: (b, 0, 0)),
            pl.BlockSpec((1, A, NN), lambda b: (b, 0, 0)),
            pl.BlockSpec((1, A, 1), lambda b: (b, 0, 0)),
            pl.BlockSpec((1, A, NN), lambda b: (b, 0, 0)),
        ] + w_specs,
        out_specs=out_specs,
        out_shape=out_shape,
        compiler_params=pltpu.CompilerParams(
            dimension_semantics=('arbitrary',),
            vmem_limit_bytes=120 * 1024 * 1024,
        ),
        interpret=interpret,
    )(R, Z.astype(jnp.int32)[..., None], N.astype(jnp.int32),
      AM[..., None], NM, *flat)
    return e3[:, 0, :1], ff, fdir


def kernel(R, Z, N, AM, NM, params):
    return _run(R, Z, N, AM, NM, params)


# speedup vs baseline: 1087.6362x; 1087.6362x over previous
import functools

import jax
import jax.numpy as jnp
import numpy as np
from jax.experimental import pallas as pl
from jax.experimental.pallas import tpu as pltpu

A, NN, F, RES, NI = 128, 48, 128, 20, 3
E = A * NN
CUTOFF = 5.0
P = 9.0
EPS = 1e-8

C = 32
CE = C * NN
NC = A // C

C_A = (P + 1.0) * (P + 2.0) / 2.0
C_B = P * (P + 2.0)
C_C = P * (P + 1.0) / 2.0
C0 = float(np.sqrt(2.0 / CUTOFF))


PREC = jax.lax.Precision.HIGHEST


def _bf(x):
    return x.astype(jnp.bfloat16).astype(jnp.float32)


def _mm(x, w):
    return jax.lax.dot_general(x, w, (((1,), (1,)), ((), ())),
                               preferred_element_type=jnp.float32)


def _mmT(x, w):
    return jax.lax.dot_general(x, w, (((1,), (0,)), ((), ())),
                               preferred_element_type=jnp.float32)


def _swish(z):
    return z * jax.nn.sigmoid(z)


def _swish_d(z):
    s = jax.nn.sigmoid(z)
    return s * (1.0 + z * (1.0 - s))


def _rep(x):
    f = x.shape[-1]
    return jnp.broadcast_to(x[:, None, :], (C, NN, f)).reshape(CE, f)


def _seg(x):
    f = x.shape[-1]
    return jnp.sum(x.reshape(C, NN, f), axis=1)


def _mul_nm(x, nm):
    f = x.shape[-1]
    return (x.reshape(C, NN, f) * nm[:, :, None]).reshape(CE, f)


def _rowsum(x):
    return jnp.sum(x, axis=1, keepdims=True)


def _newton_kernel(
    r_ref, z_ref, n_ref, am_ref, nm_ref,
    wrbf, brbf, wa0, ba0, wa1, ba1, wf, wfs0, bfs0, wfs1, bfs1,
    wr0, br0, wr1, br1, wre0, wre1, we0, be0, we1, be1,
    w1, b1, w2, b2, w3, b3, emb,
    e_ref, ff_ref, fdir_ref,
    vec3_s, v3_s, d_s, dp_s, rbf_s, cut_s,
    aS, rS0, rS1, rS2, fS0, fS1, fS2, FiS0, FiS1, FiS2,
    am_s, pr_s, gFi_s,
    gV3_s, gcut_s, grbf_s, gA_s, gR0, gR1, gR2, gF0, gF1, gF2,
    gamS, gamG, grin0, grin1, grin2, ffS, ffG, fdir_acc,
):
    rS = (rS0, rS1, rS2)
    fS = (fS0, fS1, fS2)
    FiS = (FiS0, FiS1, FiS2)
    gR = (gR0, gR1, gR2)
    gF = (gF0, gF1, gF2)
    grin = (grin0, grin1, grin2)

    Rm = r_ref[0]
    Zc = z_ref[0]
    AMc = am_ref[0]

    nvw = ((jax.lax.broadcasted_iota(jnp.int32, (1, RES), 1)
            .astype(jnp.float32) + 1.0) * (np.pi / CUTOFF))

    def chunk_onehot(k):
        Nk = n_ref[0, pl.ds(k * C, C), :]
        ids = jax.lax.broadcasted_iota(jnp.int32, (C, NN, A), 2)
        return (Nk[:, :, None] == ids).astype(jnp.float32).reshape(CE, A)

    def gat(ok, x):
        return jax.lax.dot_general(ok, x, (((1,), (0,)), ((), ())),
                                   precision=PREC,
                                   preferred_element_type=jnp.float32)

    def scat(ok, y):
        return jax.lax.dot_general(ok, y, (((0,), (0,)), ((), ())),
                                   precision=PREC,
                                   preferred_element_type=jnp.float32)

    def nm_rows(k):
        return nm_ref[0, pl.ds(k * C, C), :]

    def geom_body(k, _):
        ok = chunk_onehot(k)
        Rg = gat(ok, Rm)
        Rk = r_ref[0, pl.ds(k * C, C), :]
        vec3 = Rg - _rep(Rk)
        d2 = _rowsum(vec3 * vec3) + EPS
        d = jnp.sqrt(d2)
        dp = d + EPS
        sl = pl.ds(k * CE, CE)
        vec3_s[sl, :] = vec3
        v3_s[sl, :] = vec3 / dp
        d_s[sl, :] = d
        dp_s[sl, :] = dp
        rbf_s[sl, :] = C0 * jnp.sin(nvw * d) / dp
        xx = d / CUTOFF
        x4 = (xx * xx) * (xx * xx)
        x9 = x4 * x4 * xx
        cut_s[sl, :] = jnp.where(
            xx < 1.0,
            1.0 - C_A * x9 + C_B * (x9 * xx) - C_C * (x9 * xx * xx),
            0.0)
        return 0

    jax.lax.fori_loop(0, NC, geom_body, 0)

    zid = jax.lax.broadcasted_iota(jnp.int32, (A, 10), 1)
    a0 = jax.lax.dot_general((Zc == zid).astype(jnp.float32), emb[...],
                             (((1,), (0,)), ((), ())),
                             precision=PREC,
                             preferred_element_type=jnp.float32)
    aS[0] = a0
    zero_af = jnp.zeros((A, F), jnp.float32)
    for c in range(3):
        rS[c][0] = zero_af
        fS[c][0] = zero_af

    def fwd_layer(l, _):
        a_in = aS[l]
        za0 = _mm(a_in, wa0[l]) + ba0[l]
        am_s[...] = _mm(_swish(za0), wa1[l]) + ba1[l]
        zr0 = _mm(a_in, wr0[l]) + br0[l]
        pr_s[...] = _mm(_swish(zr0), wr1[l]) + br1[l]

        def body(k, _):
            ok = chunk_onehot(k)
            nmk = nm_rows(k)
            sl = pl.ds(k * CE, CE)
            sa = pl.ds(k * C, C)
            rbf_lin = _mm(rbf_s[sl, :], wrbf[l]) + brbf[l]
            rbf_m = rbf_lin * cut_s[sl, :]
            a_rep = _rep(am_s[sa, :])
            ag = gat(ok, am_s[...])
            msij = a_rep * ag * rbf_m
            s = _rowsum(_bf(msij) * _bf(wf[l]))
            v3 = v3_s[sl, :]
            fij3 = s * v3
            fdir_acc[l, sa, :] = jnp.sum(
                fij3.reshape(C, NN, 3) * nmk[:, :, None], axis=1)
            zfs0 = _mm(msij, wfs0[l]) + bfs0[l]
            fs = _mm(_swish(zfs0), wfs1[l]) + bfs1[l]
            zre0 = _mm(msij, wre0[l])
            pre = _mm(_swish(zre0), wre1[l])
            prk = pr_s[sa, :]
            for c in range(3):
                fi_kc = _seg(_mul_nm(fs * fij3[:, c:c + 1], nmk))
                FiS[c][l, sa, :] = fi_kc
                rg_kc = gat(ok, rS[c][l])
                drext = _seg(_mul_nm(pre * rg_kc, nmk))
                rS[c][l + 1, sa, :] = rS[c][l, sa, :] + prk * fi_kc + drext
                fS[c][l + 1, sa, :] = fS[c][l, sa, :] + fi_kc
            return 0

        jax.lax.fori_loop(0, NC, body, 0)

        de_raw = -(fS[0][l + 1] * rS[0][l + 1]
                   + fS[1][l + 1] * rS[1][l + 1]
                   + fS[2][l + 1] * rS[2][l + 1])
        ze0 = _mm(a_in, we0[l]) + be0[l]
        ee = _mm(_swish(ze0), we1[l]) + be1[l]
        aS[l + 1] = a_in + ee * de_raw
        return 0

    jax.lax.fori_loop(0, NI, fwd_layer, 0)

    a_fin = aS[NI]
    zh1 = _mm(a_fin, w1[...]) + b1[...]
    h1 = _swish(zh1)
    zh2 = _mm(h1, w2[...]) + b2[...]
    h2 = _swish(zh2)
    Ei = (_rowsum(_bf(h2) * _bf(w3[...])) + b3[0, 0]) * AMc
    Etot = jnp.sum(Ei)

    g_h2 = AMc * w3[...]
    g_h1 = _mmT(g_h2 * _swish_d(zh2), w2[...])
    gA_s[...] = _mmT(g_h1 * _swish_d(zh1), w1[...])
    for c in range(3):
        gR[c][...] = zero_af
        gF[c][...] = zero_af
    gV3_s[...] = jnp.zeros((E, 3), jnp.float32)
    gcut_s[...] = jnp.zeros((E, 1), jnp.float32)
    grbf_s[...] = jnp.zeros((E, RES), jnp.float32)

    def bwd_layer(i, _):
        l = NI - 1 - i
        a_in = aS[l]
        za0 = _mm(a_in, wa0[l]) + ba0[l]
        am_s[...] = _mm(_swish(za0), wa1[l]) + ba1[l]
        zr0 = _mm(a_in, wr0[l]) + br0[l]
        pr_s[...] = _mm(_swish(zr0), wr1[l]) + br1[l]
        ze0 = _mm(a_in, we0[l]) + be0[l]
        ee = _mm(_swish(ze0), we1[l]) + be1[l]

        de_raw = -(fS[0][l + 1] * rS[0][l + 1]
                   + fS[1][l + 1] * rS[1][l + 1]
                   + fS[2][l + 1] * rS[2][l + 1])
        g_a = gA_s[...]
        g_ee = g_a * de_raw
        g_deraw = g_a * ee
        for c in range(3):
            gF[c][...] = gF[c][...] - g_deraw * rS[c][l + 1]
            gR[c][...] = gR[c][...] - g_deraw * fS[c][l + 1]
        g_pr = (gR[0][...] * FiS[0][l] + gR[1][...] * FiS[1][l]
                + gR[2][...] * FiS[2][l])
        prv = pr_s[...]
        for c in range(3):
            gFi_s[c] = gR[c][...] * prv + gF[c][...]
            grin[c][...] = zero_af
        gamS[...] = zero_af

        def body(k, _):
            ok = chunk_onehot(k)
            nmk = nm_rows(k)
            sl = pl.ds(k * CE, CE)
            sa = pl.ds(k * C, C)
            rbf_lin = _mm(rbf_s[sl, :], wrbf[l]) + brbf[l]
            cutk = cut_s[sl, :]
            rbf_m = rbf_lin * cutk
            a_rep = _rep(am_s[sa, :])
            ag = gat(ok, am_s[...])
            msij = a_rep * ag * rbf_m
            s = _rowsum(_bf(msij) * _bf(wf[l]))
            v3 = v3_s[sl, :]
            fij3 = s * v3
            zfs0 = _mm(msij, wfs0[l]) + bfs0[l]
            fs = _mm(_swish(zfs0), wfs1[l]) + bfs1[l]
            zre0 = _mm(msij, wre0[l])
            pre = _mm(_swish(zre0), wre1[l])

            g_fs = jnp.zeros((CE, F), jnp.float32)
            g_s = jnp.zeros((CE, 1), jnp.float32)
            gv3_cols = []
            for c in range(3):
                g_fij2 = _mul_nm(_rep(gFi_s[c, pl.ds(k * C, C), :]), nmk)
                g_fs = g_fs + g_fij2 * fij3[:, c:c + 1]
                g_fij_c = _rowsum(g_fij2 * fs)
                g_s = g_s + g_fij_c * v3[:, c:c + 1]
                gv3_cols.append(g_fij_c * s)
            gV3_s[sl, :] = gV3_s[sl, :] + jnp.concatenate(gv3_cols, axis=1)

            g_pre = jnp.zeros((CE, F), jnp.float32)
            for c in range(3):
                rg_kc = gat(ok, rS[c][l])
                gdx = _mul_nm(_rep(gR[c][sa, :]), nmk)
                g_pre = g_pre + gdx * rg_kc
                grin[c][...] = grin[c][...] + scat(ok, gdx * pre)

            g_msij = _mmT(_mmT(g_pre, wre1[l]) * _swish_d(zre0), wre0[l])
            g_msij = g_msij + _mmT(
                _mmT(g_fs, wfs1[l]) * _swish_d(zfs0), wfs0[l])
            g_msij = g_msij + g_s * wf[l]

            g_arep = g_msij * ag * rbf_m
            g_ag = g_msij * a_rep * rbf_m
            g_rbfm = g_msij * a_rep * ag
            gamG[sa, :] = _seg(g_arep)
            gamS[...] = gamS[...] + scat(ok, g_ag)
            gcut_s[sl, :] = gcut_s[sl, :] + _rowsum(g_rbfm * rbf_lin)
            grbf_s[sl, :] = grbf_s[sl, :] + _mmT(g_rbfm * cutk, wrbf[l])
            return 0

        jax.lax.fori_loop(0, NC, body, 0)

        g_am = gamG[...] + gamS[...]
        g_a_in = g_a + _mmT(_mmT(g_ee, we1[l]) * _swish_d(ze0), we0[l])
        g_a_in = g_a_in + _mmT(_mmT(g_pr, wr1[l]) * _swish_d(zr0), wr0[l])
        g_a_in = g_a_in + _mmT(_mmT(g_am, wa1[l]) * _swish_d(za0), wa0[l])
        gA_s[...] = g_a_in
        for c in range(3):
            gR[c][...] = gR[c][...] + grin[c][...]
        return 0

    jax.lax.fori_loop(0, NI, bwd_layer, 0)

    ffS[...] = jnp.zeros((A, 3), jnp.float32)

    def geom_bwd(k, _):
        ok = chunk_onehot(k)
        sl = pl.ds(k * CE, CE)
        sa = pl.ds(k * C, C)
        d = d_s[sl, :]
        dp = dp_s[sl, :]
        vec3 = vec3_s[sl, :]
        grbf = grbf_s[sl, :]
        gv3 = gV3_s[sl, :]
        sin_wd = jnp.sin(nvw * d)
        cos_wd = jnp.cos(nvw * d)
        g_d = _rowsum(grbf * (C0 * nvw * cos_wd) / dp)
        g_dp = _rowsum(grbf * (-C0 * sin_wd / (dp * dp)))
        xx = d / CUTOFF
        x4 = (xx * xx) * (xx * xx)
        x8 = x4 * x4
        dcut = jnp.where(
            xx < 1.0,
            (-9.0 * C_A) * x8 + (10.0 * C_B) * (x8 * xx)
            - (11.0 * C_C) * (x8 * xx * xx),
            0.0) / CUTOFF
        g_d = g_d + gcut_s[sl, :] * dcut
        g_vec3 = gv3 / dp
        g_dp = g_dp - _rowsum(gv3 * vec3) / (dp * dp)
        g_d = g_d + g_dp
        g_vec3 = g_vec3 + g_d * vec3 / d
        ffS[...] = ffS[...] + scat(ok, g_vec3)
        ffG[sa, :] = _seg(g_vec3)
        return 0

    jax.lax.fori_loop(0, NC, geom_bwd, 0)

    e_ref[...] = jnp.broadcast_to(Etot, (1, 1, 128))
    ff_ref[...] = (ffG[...] - ffS[...])[None]
    fdir_ref[...] = (fdir_acc[0] + fdir_acc[1] + fdir_acc[2])[None]


def _stack(layers, *path):
    def get(lp):
        v = lp
        for p in path:
            v = v[p]
        return v
    out = jnp.stack([get(lp) for lp in layers])
    if out.ndim == 2:
        out = out[:, None, :]
    return out


@functools.partial(jax.jit, static_argnames=('interpret',))
def _run(R, Z, N, AM, NM, params, interpret=False):
    B = R.shape[0]
    L = params['layers']
    stacked = [
        _stack(L, 'phi_rbf', 'W'), _stack(L, 'phi_rbf', 'b'),
        _stack(L, 'phi_a', 0, 'W'), _stack(L, 'phi_a', 0, 'b'),
        _stack(L, 'phi_a', 1, 'W'), _stack(L, 'phi_a', 1, 'b'),
        _stack(L, 'phi_f', 'W'),
        _stack(L, 'phi_f_scale', 0, 'W'), _stack(L, 'phi_f_scale', 0, 'b'),
        _stack(L, 'phi_f_scale', 1, 'W'), _stack(L, 'phi_f_scale', 1, 'b'),
        _stack(L, 'phi_r', 0, 'W'), _stack(L, 'phi_r', 0, 'b'),
        _stack(L, 'phi_r', 1, 'W'), _stack(L, 'phi_r', 1, 'b'),
        _stack(L, 'phi_r_ext', 0, 'W'), _stack(L, 'phi_r_ext', 1, 'W'),
        _stack(L, 'phi_e', 0, 'W'), _stack(L, 'phi_e', 0, 'b'),
        _stack(L, 'phi_e', 1, 'W'), _stack(L, 'phi_e', 1, 'b'),
    ]
    atom = params['atomic']
    singles = [
        atom[0]['W'], atom[0]['b'].reshape(1, -1),
        atom[1]['W'], atom[1]['b'].reshape(1, -1),
        atom[2]['W'], atom[2]['b'].reshape(1, -1),
        params['emb'],
    ]
    weights = stacked + singles

    w_specs = [pl.BlockSpec(x.shape, lambda b, sh=x.shape: (0,) * len(sh))
               for x in weights]
    out_specs = [
        pl.BlockSpec((1, 1, 128), lambda b: (b, 0, 0)),
        pl.BlockSpec((1, A, 3), lambda b: (b, 0, 0)),
        pl.BlockSpec((1, A, 3), lambda b: (b, 0, 0)),
    ]
    out_shape = [
        jax.ShapeDtypeStruct((B, 1, 128), jnp.float32),
        jax.ShapeDtypeStruct((B, A, 3), jnp.float32),
        jax.ShapeDtypeStruct((B, A, 3), jnp.float32),
    ]
    vm = pltpu.VMEM
    scratch = [
        vm((E, 3), jnp.float32),
        vm((E, 3), jnp.float32),
        vm((E, 1), jnp.float32),
        vm((E, 1), jnp.float32),
        vm((E, RES), jnp.float32),
        vm((E, 1), jnp.float32),
        vm((NI + 1, A, F), jnp.float32),
    ] + [vm((NI + 1, A, F), jnp.float32) for _ in range(6)] \
      + [vm((NI, A, F), jnp.float32) for _ in range(3)] \
      + [vm((A, F), jnp.float32),
         vm((A, F), jnp.float32),
         vm((3, A, F), jnp.float32),
         vm((E, 3), jnp.float32),
         vm((E, 1), jnp.float32),
         vm((E, RES), jnp.float32),
         vm((A, F), jnp.float32)]
    scratch += [vm((A, F), jnp.float32) for _ in range(6)]
    scratch += [vm((A, F), jnp.float32),
                vm((A, F), jnp.float32)]
    scratch += [vm((A, F), jnp.float32) for _ in range(3)]
    scratch += [vm((A, 3), jnp.float32),
                vm((A, 3), jnp.float32),
                vm((NI, A, 3), jnp.float32)]

    e3, ff, fdir = pl.pallas_call(
        _newton_kernel,
        grid=(B,),
        in_specs=[
            pl.BlockSpec((1, A, 3), lambda b: (b, 0, 0)),
            pl.BlockSpec((1, A, 1), lambda b: (b, 0, 0)),
            pl.BlockSpec((1, A, NN), lambda b: (b, 0, 0)),
            pl.BlockSpec((1, A, 1), lambda b: (b, 0, 0)),
            pl.BlockSpec((1, A, NN), lambda b: (b, 0, 0)),
        ] + w_specs,
        out_specs=out_specs,
        out_shape=out_shape,
        scratch_shapes=scratch,
        compiler_params=pltpu.CompilerParams(
            dimension_semantics=('arbitrary',),
            vmem_limit_bytes=100 * 1024 * 1024,
        ),
        interpret=interpret,
    )(R, Z.astype(jnp.int32)[..., None], N.astype(jnp.int32),
      AM[..., None], NM, *weights)
    return e3[:, 0, :1], ff, fdir


def kernel(R, Z, N, AM, NM, params):
    return _run(R, Z, N, AM, NM, params)


# speedup vs baseline: 1591.2054x; 1.4630x over previous
import functools

import jax
import jax.numpy as jnp
import numpy as np
from jax.experimental import pallas as pl
from jax.experimental.pallas import tpu as pltpu

A, NN, F, RES, NI = 128, 48, 128, 20, 3
E = A * NN
CUTOFF = 5.0
P = 9.0
EPS = 1e-8

C = 32
CE = C * NN
NC = A // C

C_A = (P + 1.0) * (P + 2.0) / 2.0
C_B = P * (P + 2.0)
C_C = P * (P + 1.0) / 2.0
C0 = float(np.sqrt(2.0 / CUTOFF))


PREC = jax.lax.Precision.HIGHEST


def _bf(x):
    return x.astype(jnp.bfloat16).astype(jnp.float32)


def _mm(x, w):
    return jax.lax.dot_general(x, w, (((1,), (1,)), ((), ())),
                               preferred_element_type=jnp.float32)


def _mmT(x, w):
    return jax.lax.dot_general(x, w, (((1,), (0,)), ((), ())),
                               preferred_element_type=jnp.float32)


def _swish(z):
    return z * jax.nn.sigmoid(z)


def _swish_d(z):
    s = jax.nn.sigmoid(z)
    return s * (1.0 + z * (1.0 - s))


def _rep(x):
    f = x.shape[-1]
    return jnp.broadcast_to(x[:, None, :], (C, NN, f)).reshape(CE, f)


def _seg(x):
    f = x.shape[-1]
    return jnp.sum(x.reshape(C, NN, f), axis=1)


def _mul_nm(x, nm):
    f = x.shape[-1]
    return (x.reshape(C, NN, f) * nm[:, :, None]).reshape(CE, f)


def _rowsum(x):
    return jnp.sum(x, axis=1, keepdims=True)


def _newton_kernel(
    r_ref, z_ref, n_ref, am_ref, nm_ref,
    wrbf, brbf, wa0, ba0, wa1, ba1, wf, wfs0, bfs0, wfs1, bfs1,
    wr0, br0, wr1, br1, wre0, wre1, we0, be0, we1, be1,
    w1, b1, w2, b2, w3, b3, emb,
    e_ref, ff_ref, fdir_ref,
    vec3_s, v3_s, d_s, dp_s, rbf_s, cut_s,
    aS, rS0, rS1, rS2, fS0, fS1, fS2, FiS0, FiS1, FiS2,
    am_s, pr_s, gFi_s,
    gV3_s, gcut_s, grbf_s, gA_s, gR0, gR1, gR2, gF0, gF1, gF2,
    gamS, gamG, grin0, grin1, grin2, ffS, ffG, fdir_acc,
):
    rS = (rS0, rS1, rS2)
    fS = (fS0, fS1, fS2)
    FiS = (FiS0, FiS1, FiS2)
    gR = (gR0, gR1, gR2)
    gF = (gF0, gF1, gF2)
    grin = (grin0, grin1, grin2)

    Rm = r_ref[0]
    Zc = z_ref[0]
    AMc = am_ref[0]

    nvw = ((jax.lax.broadcasted_iota(jnp.int32, (1, RES), 1)
            .astype(jnp.float32) + 1.0) * (np.pi / CUTOFF))

    def chunk_onehot(k):
        Nk = n_ref[0, pl.ds(k * C, C), :]
        ids = jax.lax.broadcasted_iota(jnp.int32, (C, NN, A), 2)
        return (Nk[:, :, None] == ids).astype(jnp.bfloat16).reshape(CE, A)

    def _split3(x):
        h = x.astype(jnp.bfloat16)
        r = x - h.astype(jnp.float32)
        m = r.astype(jnp.bfloat16)
        lo = (r - m.astype(jnp.float32)).astype(jnp.bfloat16)
        return h, m, lo

    def _dotn(a, b, dims):
        return jax.lax.dot_general(a, b, dims,
                                   preferred_element_type=jnp.float32)

    GAT_D = (((1,), (0,)), ((), ()))
    SCAT_D = (((0,), (0,)), ((), ()))

    def gat(ok, x):
        h, m, lo = _split3(x)
        return (_dotn(ok, h, GAT_D) + _dotn(ok, m, GAT_D)
                + _dotn(ok, lo, GAT_D))

    def scat(ok, y):
        h, m, lo = _split3(y)
        return (_dotn(ok, h, SCAT_D) + _dotn(ok, m, SCAT_D)
                + _dotn(ok, lo, SCAT_D))

    def nm_rows(k):
        return nm_ref[0, pl.ds(k * C, C), :]

    def geom_body(k, _):
        ok = chunk_onehot(k)
        Rg = gat(ok, Rm)
        Rk = r_ref[0, pl.ds(k * C, C), :]
        vec3 = Rg - _rep(Rk)
        d2 = _rowsum(vec3 * vec3) + EPS
        d = jnp.sqrt(d2)
        dp = d + EPS
        sl = pl.ds(k * CE, CE)
        vec3_s[sl, :] = vec3
        v3_s[sl, :] = vec3 / dp
        d_s[sl, :] = d
        dp_s[sl, :] = dp
        rbf_s[sl, :] = C0 * jnp.sin(nvw * d) / dp
        xx = d / CUTOFF
        x4 = (xx * xx) * (xx * xx)
        x9 = x4 * x4 * xx
        cut_s[sl, :] = jnp.where(
            xx < 1.0,
            1.0 - C_A * x9 + C_B * (x9 * xx) - C_C * (x9 * xx * xx),
            0.0)
        return 0

    jax.lax.fori_loop(0, NC, geom_body, 0)

    zid = jax.lax.broadcasted_iota(jnp.int32, (A, 10), 1)
    a0 = jax.lax.dot_general((Zc == zid).astype(jnp.float32), emb[...],
                             (((1,), (0,)), ((), ())),
                             precision=PREC,
                             preferred_element_type=jnp.float32)
    aS[0] = a0
    zero_af = jnp.zeros((A, F), jnp.float32)
    for c in range(3):
        rS[c][0] = zero_af
        fS[c][0] = zero_af

    def fwd_layer(l, _):
        a_in = aS[l]
        za0 = _mm(a_in, wa0[l]) + ba0[l]
        am_s[...] = _mm(_swish(za0), wa1[l]) + ba1[l]
        zr0 = _mm(a_in, wr0[l]) + br0[l]
        pr_s[...] = _mm(_swish(zr0), wr1[l]) + br1[l]

        def body(k, _):
            ok = chunk_onehot(k)
            nmk = nm_rows(k)
            sl = pl.ds(k * CE, CE)
            sa = pl.ds(k * C, C)
            rbf_lin = _mm(rbf_s[sl, :], wrbf[l]) + brbf[l]
            rbf_m = rbf_lin * cut_s[sl, :]
            a_rep = _rep(am_s[sa, :])
            ag = gat(ok, am_s[...])
            msij = a_rep * ag * rbf_m
            s = _rowsum(_bf(msij) * _bf(wf[l]))
            v3 = v3_s[sl, :]
            fij3 = s * v3
            fdir_acc[l, sa, :] = jnp.sum(
                fij3.reshape(C, NN, 3) * nmk[:, :, None], axis=1)
            zfs0 = _mm(msij, wfs0[l]) + bfs0[l]
            fs = _mm(_swish(zfs0), wfs1[l]) + bfs1[l]
            zre0 = _mm(msij, wre0[l])
            pre = _mm(_swish(zre0), wre1[l])
            prk = pr_s[sa, :]
            for c in range(3):
                fi_kc = _seg(_mul_nm(fs * fij3[:, c:c + 1], nmk))
                FiS[c][l, sa, :] = fi_kc
                rg_kc = gat(ok, rS[c][l])
                drext = _seg(_mul_nm(pre * rg_kc, nmk))
                rS[c][l + 1, sa, :] = rS[c][l, sa, :] + prk * fi_kc + drext
                fS[c][l + 1, sa, :] = fS[c][l, sa, :] + fi_kc
            return 0

        jax.lax.fori_loop(0, NC, body, 0)

        de_raw = -(fS[0][l + 1] * rS[0][l + 1]
                   + fS[1][l + 1] * rS[1][l + 1]
                   + fS[2][l + 1] * rS[2][l + 1])
        ze0 = _mm(a_in, we0[l]) + be0[l]
        ee = _mm(_swish(ze0), we1[l]) + be1[l]
        aS[l + 1] = a_in + ee * de_raw
        return 0

    jax.lax.fori_loop(0, NI, fwd_layer, 0)

    a_fin = aS[NI]
    zh1 = _mm(a_fin, w1[...]) + b1[...]
    h1 = _swish(zh1)
    zh2 = _mm(h1, w2[...]) + b2[...]
    h2 = _swish(zh2)
    Ei = (_rowsum(_bf(h2) * _bf(w3[...])) + b3[0, 0]) * AMc
    Etot = jnp.sum(Ei)

    g_h2 = AMc * w3[...]
    g_h1 = _mmT(g_h2 * _swish_d(zh2), w2[...])
    gA_s[...] = _mmT(g_h1 * _swish_d(zh1), w1[...])
    for c in range(3):
        gR[c][...] = zero_af
        gF[c][...] = zero_af
    gV3_s[...] = jnp.zeros((E, 3), jnp.float32)
    gcut_s[...] = jnp.zeros((E, 1), jnp.float32)
    grbf_s[...] = jnp.zeros((E, RES), jnp.float32)

    def bwd_layer(i, _):
        l = NI - 1 - i
        a_in = aS[l]
        za0 = _mm(a_in, wa0[l]) + ba0[l]
        am_s[...] = _mm(_swish(za0), wa1[l]) + ba1[l]
        zr0 = _mm(a_in, wr0[l]) + br0[l]
        pr_s[...] = _mm(_swish(zr0), wr1[l]) + br1[l]
        ze0 = _mm(a_in, we0[l]) + be0[l]
        ee = _mm(_swish(ze0), we1[l]) + be1[l]

        de_raw = -(fS[0][l + 1] * rS[0][l + 1]
                   + fS[1][l + 1] * rS[1][l + 1]
                   + fS[2][l + 1] * rS[2][l + 1])
        g_a = gA_s[...]
        g_ee = g_a * de_raw
        g_deraw = g_a * ee
        for c in range(3):
            gF[c][...] = gF[c][...] - g_deraw * rS[c][l + 1]
            gR[c][...] = gR[c][...] - g_deraw * fS[c][l + 1]
        g_pr = (gR[0][...] * FiS[0][l] + gR[1][...] * FiS[1][l]
                + gR[2][...] * FiS[2][l])
        prv = pr_s[...]
        for c in range(3):
            gFi_s[c] = gR[c][...] * prv + gF[c][...]
            grin[c][...] = zero_af
        gamS[...] = zero_af

        def body(k, _):
            ok = chunk_onehot(k)
            nmk = nm_rows(k)
            sl = pl.ds(k * CE, CE)
            sa = pl.ds(k * C, C)
            rbf_lin = _mm(rbf_s[sl, :], wrbf[l]) + brbf[l]
            cutk = cut_s[sl, :]
            rbf_m = rbf_lin * cutk
            a_rep = _rep(am_s[sa, :])
            ag = gat(ok, am_s[...])
            msij = a_rep * ag * rbf_m
            s = _rowsum(_bf(msij) * _bf(wf[l]))
            v3 = v3_s[sl, :]
            fij3 = s * v3
            zfs0 = _mm(msij, wfs0[l]) + bfs0[l]
            fs = _mm(_swish(zfs0), wfs1[l]) + bfs1[l]
            zre0 = _mm(msij, wre0[l])
            pre = _mm(_swish(zre0), wre1[l])

            g_fs = jnp.zeros((CE, F), jnp.float32)
            g_s = jnp.zeros((CE, 1), jnp.float32)
            gv3_cols = []
            for c in range(3):
                g_fij2 = _mul_nm(_rep(gFi_s[c, pl.ds(k * C, C), :]), nmk)
                g_fs = g_fs + g_fij2 * fij3[:, c:c + 1]
                g_fij_c = _rowsum(g_fij2 * fs)
                g_s = g_s + g_fij_c * v3[:, c:c + 1]
                gv3_cols.append(g_fij_c * s)
            gV3_s[sl, :] = gV3_s[sl, :] + jnp.concatenate(gv3_cols, axis=1)

            g_pre = jnp.zeros((CE, F), jnp.float32)
            for c in range(3):
                rg_kc = gat(ok, rS[c][l])
                gdx = _mul_nm(_rep(gR[c][sa, :]), nmk)
                g_pre = g_pre + gdx * rg_kc
                grin[c][...] = grin[c][...] + scat(ok, gdx * pre)

            g_msij = _mmT(_mmT(g_pre, wre1[l]) * _swish_d(zre0), wre0[l])
            g_msij = g_msij + _mmT(
                _mmT(g_fs, wfs1[l]) * _swish_d(zfs0), wfs0[l])
            g_msij = g_msij + g_s * wf[l]

            g_arep = g_msij * ag * rbf_m
            g_ag = g_msij * a_rep * rbf_m
            g_rbfm = g_msij * a_rep * ag
            gamG[sa, :] = _seg(g_arep)
            gamS[...] = gamS[...] + scat(ok, g_ag)
            gcut_s[sl, :] = gcut_s[sl, :] + _rowsum(g_rbfm * rbf_lin)
            grbf_s[sl, :] = grbf_s[sl, :] + _mmT(g_rbfm * cutk, wrbf[l])
            return 0

        jax.lax.fori_loop(0, NC, body, 0)

        g_am = gamG[...] + gamS[...]
        g_a_in = g_a + _mmT(_mmT(g_ee, we1[l]) * _swish_d(ze0), we0[l])
        g_a_in = g_a_in + _mmT(_mmT(g_pr, wr1[l]) * _swish_d(zr0), wr0[l])
        g_a_in = g_a_in + _mmT(_mmT(g_am, wa1[l]) * _swish_d(za0), wa0[l])
        gA_s[...] = g_a_in
        for c in range(3):
            gR[c][...] = gR[c][...] + grin[c][...]
        return 0

    jax.lax.fori_loop(0, NI, bwd_layer, 0)

    ffS[...] = jnp.zeros((A, 3), jnp.float32)

    def geom_bwd(k, _):
        ok = chunk_onehot(k)
        sl = pl.ds(k * CE, CE)
        sa = pl.ds(k * C, C)
        d = d_s[sl, :]
        dp = dp_s[sl, :]
        vec3 = vec3_s[sl, :]
        grbf = grbf_s[sl, :]
        gv3 = gV3_s[sl, :]
        sin_wd = jnp.sin(nvw * d)
        cos_wd = jnp.cos(nvw * d)
        g_d = _rowsum(grbf * (C0 * nvw * cos_wd) / dp)
        g_dp = _rowsum(grbf * (-C0 * sin_wd / (dp * dp)))
        xx = d / CUTOFF
        x4 = (xx * xx) * (xx * xx)
        x8 = x4 * x4
        dcut = jnp.where(
            xx < 1.0,
            (-9.0 * C_A) * x8 + (10.0 * C_B) * (x8 * xx)
            - (11.0 * C_C) * (x8 * xx * xx),
            0.0) / CUTOFF
        g_d = g_d + gcut_s[sl, :] * dcut
        g_vec3 = gv3 / dp
        g_dp = g_dp - _rowsum(gv3 * vec3) / (dp * dp)
        g_d = g_d + g_dp
        g_vec3 = g_vec3 + g_d * vec3 / d
        ffS[...] = ffS[...] + scat(ok, g_vec3)
        ffG[sa, :] = _seg(g_vec3)
        return 0

    jax.lax.fori_loop(0, NC, geom_bwd, 0)

    e_ref[...] = jnp.broadcast_to(Etot, (1, 1, 128))
    ff_ref[...] = (ffG[...] - ffS[...])[None]
    fdir_ref[...] = (fdir_acc[0] + fdir_acc[1] + fdir_acc[2])[None]


def _stack(layers, *path):
    def get(lp):
        v = lp
        for p in path:
            v = v[p]
        return v
    out = jnp.stack([get(lp) for lp in layers])
    if out.ndim == 2:
        out = out[:, None, :]
    return out


@functools.partial(jax.jit, static_argnames=('interpret',))
def _run(R, Z, N, AM, NM, params, interpret=False):
    B = R.shape[0]
    L = params['layers']
    stacked = [
        _stack(L, 'phi_rbf', 'W'), _stack(L, 'phi_rbf', 'b'),
        _stack(L, 'phi_a', 0, 'W'), _stack(L, 'phi_a', 0, 'b'),
        _stack(L, 'phi_a', 1, 'W'), _stack(L, 'phi_a', 1, 'b'),
        _stack(L, 'phi_f', 'W'),
        _stack(L, 'phi_f_scale', 0, 'W'), _stack(L, 'phi_f_scale', 0, 'b'),
        _stack(L, 'phi_f_scale', 1, 'W'), _stack(L, 'phi_f_scale', 1, 'b'),
        _stack(L, 'phi_r', 0, 'W'), _stack(L, 'phi_r', 0, 'b'),
        _stack(L, 'phi_r', 1, 'W'), _stack(L, 'phi_r', 1, 'b'),
        _stack(L, 'phi_r_ext', 0, 'W'), _stack(L, 'phi_r_ext', 1, 'W'),
        _stack(L, 'phi_e', 0, 'W'), _stack(L, 'phi_e', 0, 'b'),
        _stack(L, 'phi_e', 1, 'W'), _stack(L, 'phi_e', 1, 'b'),
    ]
    atom = params['atomic']
    singles = [
        atom[0]['W'], atom[0]['b'].reshape(1, -1),
        atom[1]['W'], atom[1]['b'].reshape(1, -1),
        atom[2]['W'], atom[2]['b'].reshape(1, -1),
        params['emb'],
    ]
    weights = stacked + singles

    w_specs = [pl.BlockSpec(x.shape, lambda b, sh=x.shape: (0,) * len(sh))
               for x in weights]
    out_specs = [
        pl.BlockSpec((1, 1, 128), lambda b: (b, 0, 0)),
        pl.BlockSpec((1, A, 3), lambda b: (b, 0, 0)),
        pl.BlockSpec((1, A, 3), lambda b: (b, 0, 0)),
    ]
    out_shape = [
        jax.ShapeDtypeStruct((B, 1, 128), jnp.float32),
        jax.ShapeDtypeStruct((B, A, 3), jnp.float32),
        jax.ShapeDtypeStruct((B, A, 3), jnp.float32),
    ]
    vm = pltpu.VMEM
    scratch = [
        vm((E, 3), jnp.float32),
        vm((E, 3), jnp.float32),
        vm((E, 1), jnp.float32),
        vm((E, 1), jnp.float32),
        vm((E, RES), jnp.float32),
        vm((E, 1), jnp.float32),
        vm((NI + 1, A, F), jnp.float32),
    ] + [vm((NI + 1, A, F), jnp.float32) for _ in range(6)] \
      + [vm((NI, A, F), jnp.float32) for _ in range(3)] \
      + [vm((A, F), jnp.float32),
         vm((A, F), jnp.float32),
         vm((3, A, F), jnp.float32),
         vm((E, 3), jnp.float32),
         vm((E, 1), jnp.float32),
         vm((E, RES), jnp.float32),
         vm((A, F), jnp.float32)]
    scratch += [vm((A, F), jnp.float32) for _ in range(6)]
    scratch += [vm((A, F), jnp.float32),
                vm((A, F), jnp.float32)]
    scratch += [vm((A, F), jnp.float32) for _ in range(3)]
    scratch += [vm((A, 3), jnp.float32),
                vm((A, 3), jnp.float32),
                vm((NI, A, 3), jnp.float32)]

    e3, ff, fdir = pl.pallas_call(
        _newton_kernel,
        grid=(B,),
        in_specs=[
            pl.BlockSpec((1, A, 3), lambda b: (b, 0, 0)),
            pl.BlockSpec((1, A, 1), lambda b: (b, 0, 0)),
            pl.BlockSpec((1, A, NN), lambda b: (b, 0, 0)),
            pl.BlockSpec((1, A, 1), lambda b: (b, 0, 0)),
            pl.BlockSpec((1, A, NN), lambda b: (b, 0, 0)),
        ] + w_specs,
        out_specs=out_specs,
        out_shape=out_shape,
        scratch_shapes=scratch,
        compiler_params=pltpu.CompilerParams(
            dimension_semantics=('arbitrary',),
            vmem_limit_bytes=100 * 1024 * 1024,
        ),
        interpret=interpret,
    )(R, Z.astype(jnp.int32)[..., None], N.astype(jnp.int32),
      AM[..., None], NM, *weights)
    return e3[:, 0, :1], ff, fdir


def kernel(R, Z, N, AM, NM, params):
    return _run(R, Z, N, AM, NM, params)


# speedup vs baseline: 1719.5780x; 1.0807x over previous
import functools

import jax
import jax.numpy as jnp
import numpy as np
from jax.experimental import pallas as pl
from jax.experimental.pallas import tpu as pltpu

A, NN, F, RES, NI = 128, 48, 128, 20, 3
E = A * NN
CUTOFF = 5.0
P = 9.0
EPS = 1e-8

C = 32
CE = C * NN
NC = A // C

C_A = (P + 1.0) * (P + 2.0) / 2.0
C_B = P * (P + 2.0)
C_C = P * (P + 1.0) / 2.0
C0 = float(np.sqrt(2.0 / CUTOFF))


PREC = jax.lax.Precision.HIGHEST


def _bf(x):
    return x.astype(jnp.bfloat16).astype(jnp.float32)


def _mm(x, w):
    return jax.lax.dot_general(x, w, (((1,), (1,)), ((), ())),
                               preferred_element_type=jnp.float32)


def _mmT(x, w):
    return jax.lax.dot_general(x, w, (((1,), (0,)), ((), ())),
                               preferred_element_type=jnp.float32)


def _swish(z):
    return z * jax.nn.sigmoid(z)


def _swish_d(z):
    s = jax.nn.sigmoid(z)
    return s * (1.0 + z * (1.0 - s))


def _rep(x):
    f = x.shape[-1]
    return jnp.broadcast_to(x[:, None, :], (C, NN, f)).reshape(CE, f)


def _seg(x):
    f = x.shape[-1]
    return jnp.sum(x.reshape(C, NN, f), axis=1)


def _mul_nm(x, nm):
    f = x.shape[-1]
    return (x.reshape(C, NN, f) * nm[:, :, None]).reshape(CE, f)


def _rowsum(x):
    return jnp.sum(x, axis=1, keepdims=True)


def _newton_kernel(
    r_ref, z_ref, n_ref, am_ref, nm_ref,
    wrbf, brbf, wa0, ba0, wa1, ba1, wf, wfs0, bfs0, wfs1, bfs1,
    wr0, br0, wr1, br1, wre0, wre1, we0, be0, we1, be1,
    w1, b1, w2, b2, w3, b3, emb,
    e_ref, ff_ref, fdir_ref,
    vec3_s, v3_s, d_s, rdp_s, cos_s, rbf_s, cut_s,
    aS, rS0, rS1, rS2, fS0, fS1, fS2, FiS0, FiS1, FiS2,
    am_s, pr_s, gFi_s,
    gV3_s, gcut_s, grbf_s, gA_s, gR0, gR1, gR2, gF0, gF1, gF2,
    gamS, gamG, grin0, grin1, grin2, ffS, ffG, fdir_acc,
):
    rS = (rS0, rS1, rS2)
    fS = (fS0, fS1, fS2)
    FiS = (FiS0, FiS1, FiS2)
    gR = (gR0, gR1, gR2)
    gF = (gF0, gF1, gF2)
    grin = (grin0, grin1, grin2)

    Rm = r_ref[0]
    Zc = z_ref[0]
    AMc = am_ref[0]

    nvw = ((jax.lax.broadcasted_iota(jnp.int32, (1, RES), 1)
            .astype(jnp.float32) + 1.0) * (np.pi / CUTOFF))

    def chunk_onehot(k):
        Nk = n_ref[0, pl.ds(k * C, C), :]
        ids = jax.lax.broadcasted_iota(jnp.int32, (C, NN, A), 2)
        return (Nk[:, :, None] == ids).astype(jnp.bfloat16).reshape(CE, A)

    def _split3(x):
        h = x.astype(jnp.bfloat16)
        r = x - h.astype(jnp.float32)
        m = r.astype(jnp.bfloat16)
        lo = (r - m.astype(jnp.float32)).astype(jnp.bfloat16)
        return h, m, lo

    def _dotn(a, b, dims):
        return jax.lax.dot_general(a, b, dims,
                                   preferred_element_type=jnp.float32)

    GAT_D = (((1,), (0,)), ((), ()))
    SCAT_D = (((0,), (0,)), ((), ()))

    def gat(ok, x):
        h, m, lo = _split3(x)
        return (_dotn(ok, h, GAT_D) + _dotn(ok, m, GAT_D)
                + _dotn(ok, lo, GAT_D))

    def scat(ok, y):
        h, m, lo = _split3(y)
        return (_dotn(ok, h, SCAT_D) + _dotn(ok, m, SCAT_D)
                + _dotn(ok, lo, SCAT_D))

    def nm_rows(k):
        return nm_ref[0, pl.ds(k * C, C), :]

    def geom_body(k, _):
        ok = chunk_onehot(k)
        Rg = gat(ok, Rm)
        Rk = r_ref[0, pl.ds(k * C, C), :]
        vec3 = Rg - _rep(Rk)
        d2 = _rowsum(vec3 * vec3) + EPS
        d = jnp.sqrt(d2)
        rdp = 1.0 / (d + EPS)
        sl = pl.ds(k * CE, CE)
        vec3_s[sl, :] = vec3
        v3_s[sl, :] = vec3 * rdp
        d_s[sl, :] = d
        rdp_s[sl, :] = rdp
        cos_s[sl, :] = jnp.cos(nvw * d)
        rbf_s[sl, :] = (C0 * jnp.sin(nvw * d)) * rdp
        xx = d / CUTOFF
        x4 = (xx * xx) * (xx * xx)
        x9 = x4 * x4 * xx
        cut_s[sl, :] = jnp.where(
            xx < 1.0,
            1.0 - C_A * x9 + C_B * (x9 * xx) - C_C * (x9 * xx * xx),
            0.0)
        return 0

    jax.lax.fori_loop(0, NC, geom_body, 0)

    zid = jax.lax.broadcasted_iota(jnp.int32, (A, 10), 1)
    a0 = jax.lax.dot_general((Zc == zid).astype(jnp.float32), emb[...],
                             (((1,), (0,)), ((), ())),
                             precision=PREC,
                             preferred_element_type=jnp.float32)
    aS[0] = a0
    zero_af = jnp.zeros((A, F), jnp.float32)
    for c in range(3):
        rS[c][0] = zero_af
        fS[c][0] = zero_af

    def fwd_layer(l, _):
        a_in = aS[l]
        za0 = _mm(a_in, wa0[l]) + ba0[l]
        am_s[...] = _mm(_swish(za0), wa1[l]) + ba1[l]
        zr0 = _mm(a_in, wr0[l]) + br0[l]
        pr_s[...] = _mm(_swish(zr0), wr1[l]) + br1[l]

        def body(k, _):
            ok = chunk_onehot(k)
            nmk = nm_rows(k)
            sl = pl.ds(k * CE, CE)
            sa = pl.ds(k * C, C)
            rbf_lin = _mm(rbf_s[sl, :], wrbf[l]) + brbf[l]
            rbf_m = rbf_lin * cut_s[sl, :]
            a_rep = _rep(am_s[sa, :])
            ag = gat(ok, am_s[...])
            msij = a_rep * ag * rbf_m
            s = _rowsum(_bf(msij) * _bf(wf[l]))
            v3 = v3_s[sl, :]
            fij3 = s * v3
            fdir_acc[l, sa, :] = jnp.sum(
                fij3.reshape(C, NN, 3) * nmk[:, :, None], axis=1)
            zfs0 = _mm(msij, wfs0[l]) + bfs0[l]
            fs = _mm(_swish(zfs0), wfs1[l]) + bfs1[l]
            zre0 = _mm(msij, wre0[l])
            pre = _mm(_swish(zre0), wre1[l])
            prk = pr_s[sa, :]
            for c in range(3):
                fi_kc = _seg(_mul_nm(fs * fij3[:, c:c + 1], nmk))
                FiS[c][l, sa, :] = fi_kc
                rg_kc = gat(ok, rS[c][l])
                drext = _seg(_mul_nm(pre * rg_kc, nmk))
                rS[c][l + 1, sa, :] = rS[c][l, sa, :] + prk * fi_kc + drext
                fS[c][l + 1, sa, :] = fS[c][l, sa, :] + fi_kc
            return 0

        jax.lax.fori_loop(0, NC, body, 0)

        de_raw = -(fS[0][l + 1] * rS[0][l + 1]
                   + fS[1][l + 1] * rS[1][l + 1]
                   + fS[2][l + 1] * rS[2][l + 1])
        ze0 = _mm(a_in, we0[l]) + be0[l]
        ee = _mm(_swish(ze0), we1[l]) + be1[l]
        aS[l + 1] = a_in + ee * de_raw
        return 0

    jax.lax.fori_loop(0, NI, fwd_layer, 0)

    a_fin = aS[NI]
    zh1 = _mm(a_fin, w1[...]) + b1[...]
    h1 = _swish(zh1)
    zh2 = _mm(h1, w2[...]) + b2[...]
    h2 = _swish(zh2)
    Ei = (_rowsum(_bf(h2) * _bf(w3[...])) + b3[0, 0]) * AMc
    Etot = jnp.sum(Ei)

    g_h2 = AMc * w3[...]
    g_h1 = _mmT(g_h2 * _swish_d(zh2), w2[...])
    gA_s[...] = _mmT(g_h1 * _swish_d(zh1), w1[...])
    for c in range(3):
        gR[c][...] = zero_af
        gF[c][...] = zero_af
    gV3_s[...] = jnp.zeros((E, 3), jnp.float32)
    gcut_s[...] = jnp.zeros((E, 1), jnp.float32)
    grbf_s[...] = jnp.zeros((E, RES), jnp.float32)

    def bwd_layer(i, _):
        l = NI - 1 - i
        a_in = aS[l]
        za0 = _mm(a_in, wa0[l]) + ba0[l]
        am_s[...] = _mm(_swish(za0), wa1[l]) + ba1[l]
        zr0 = _mm(a_in, wr0[l]) + br0[l]
        pr_s[...] = _mm(_swish(zr0), wr1[l]) + br1[l]
        ze0 = _mm(a_in, we0[l]) + be0[l]
        ee = _mm(_swish(ze0), we1[l]) + be1[l]

        de_raw = -(fS[0][l + 1] * rS[0][l + 1]
                   + fS[1][l + 1] * rS[1][l + 1]
                   + fS[2][l + 1] * rS[2][l + 1])
        g_a = gA_s[...]
        g_ee = g_a * de_raw
        g_deraw = g_a * ee
        for c in range(3):
            gF[c][...] = gF[c][...] - g_deraw * rS[c][l + 1]
            gR[c][...] = gR[c][...] - g_deraw * fS[c][l + 1]
        g_pr = (gR[0][...] * FiS[0][l] + gR[1][...] * FiS[1][l]
                + gR[2][...] * FiS[2][l])
        prv = pr_s[...]
        for c in range(3):
            gFi_s[c] = gR[c][...] * prv + gF[c][...]
            grin[c][...] = zero_af
        gamS[...] = zero_af

        def body(k, _):
            ok = chunk_onehot(k)
            nmk = nm_rows(k)
            sl = pl.ds(k * CE, CE)
            sa = pl.ds(k * C, C)
            rbf_lin = _mm(rbf_s[sl, :], wrbf[l]) + brbf[l]
            cutk = cut_s[sl, :]
            rbf_m = rbf_lin * cutk
            a_rep = _rep(am_s[sa, :])
            ag = gat(ok, am_s[...])
            msij = a_rep * ag * rbf_m
            s = _rowsum(_bf(msij) * _bf(wf[l]))
            v3 = v3_s[sl, :]
            fij3 = s * v3
            zfs0 = _mm(msij, wfs0[l]) + bfs0[l]
            fs = _mm(_swish(zfs0), wfs1[l]) + bfs1[l]
            zre0 = _mm(msij, wre0[l])
            pre = _mm(_swish(zre0), wre1[l])

            g_fs = jnp.zeros((CE, F), jnp.float32)
            g_s = jnp.zeros((CE, 1), jnp.float32)
            gv3_cols = []
            for c in range(3):
                g_fij2 = _mul_nm(_rep(gFi_s[c, pl.ds(k * C, C), :]), nmk)
                g_fs = g_fs + g_fij2 * fij3[:, c:c + 1]
                g_fij_c = _rowsum(g_fij2 * fs)
                g_s = g_s + g_fij_c * v3[:, c:c + 1]
                gv3_cols.append(g_fij_c * s)
            gV3_s[sl, :] = gV3_s[sl, :] + jnp.concatenate(gv3_cols, axis=1)

            g_pre = jnp.zeros((CE, F), jnp.float32)
            for c in range(3):
                rg_kc = gat(ok, rS[c][l])
                gdx = _mul_nm(_rep(gR[c][sa, :]), nmk)
                g_pre = g_pre + gdx * rg_kc
                grin[c][...] = grin[c][...] + scat(ok, gdx * pre)

            g_msij = _mmT(_mmT(g_pre, wre1[l]) * _swish_d(zre0), wre0[l])
            g_msij = g_msij + _mmT(
                _mmT(g_fs, wfs1[l]) * _swish_d(zfs0), wfs0[l])
            g_msij = g_msij + g_s * wf[l]

            g_arep = g_msij * ag * rbf_m
            g_ag = g_msij * a_rep * rbf_m
            g_rbfm = g_msij * a_rep * ag
            gamG[sa, :] = _seg(g_arep)
            gamS[...] = gamS[...] + scat(ok, g_ag)
            gcut_s[sl, :] = gcut_s[sl, :] + _rowsum(g_rbfm * rbf_lin)
            grbf_s[sl, :] = grbf_s[sl, :] + _mmT(g_rbfm * cutk, wrbf[l])
            return 0

        jax.lax.fori_loop(0, NC, body, 0)

        g_am = gamG[...] + gamS[...]
        g_a_in = g_a + _mmT(_mmT(g_ee, we1[l]) * _swish_d(ze0), we0[l])
        g_a_in = g_a_in + _mmT(_mmT(g_pr, wr1[l]) * _swish_d(zr0), wr0[l])
        g_a_in = g_a_in + _mmT(_mmT(g_am, wa1[l]) * _swish_d(za0), wa0[l])
        gA_s[...] = g_a_in
        for c in range(3):
            gR[c][...] = gR[c][...] + grin[c][...]
        return 0

    jax.lax.fori_loop(0, NI, bwd_layer, 0)

    ffS[...] = jnp.zeros((A, 3), jnp.float32)

    def geom_bwd(k, _):
        ok = chunk_onehot(k)
        sl = pl.ds(k * CE, CE)
        sa = pl.ds(k * C, C)
        d = d_s[sl, :]
        rdp = rdp_s[sl, :]
        vec3 = vec3_s[sl, :]
        grbf = grbf_s[sl, :]
        gv3 = gV3_s[sl, :]
        g_d = _rowsum(grbf * ((C0 * nvw) * cos_s[sl, :])) * rdp
        g_dp = -_rowsum(grbf * rbf_s[sl, :]) * rdp
        xx = d / CUTOFF
        x4 = (xx * xx) * (xx * xx)
        x8 = x4 * x4
        dcut = jnp.where(
            xx < 1.0,
            (-9.0 * C_A) * x8 + (10.0 * C_B) * (x8 * xx)
            - (11.0 * C_C) * (x8 * xx * xx),
            0.0) / CUTOFF
        g_d = g_d + gcut_s[sl, :] * dcut
        g_vec3 = gv3 * rdp
        g_dp = g_dp - _rowsum(gv3 * vec3) * (rdp * rdp)
        g_d = g_d + g_dp
        g_vec3 = g_vec3 + (g_d / d) * vec3
        ffS[...] = ffS[...] + scat(ok, g_vec3)
        ffG[sa, :] = _seg(g_vec3)
        return 0

    jax.lax.fori_loop(0, NC, geom_bwd, 0)

    e_ref[...] = jnp.broadcast_to(Etot, (1, 1, 128))
    ff_ref[...] = (ffG[...] - ffS[...])[None]
    fdir_ref[...] = (fdir_acc[0] + fdir_acc[1] + fdir_acc[2])[None]


def _stack(layers, *path):
    def get(lp):
        v = lp
        for p in path:
            v = v[p]
        return v
    out = jnp.stack([get(lp) for lp in layers])
    if out.ndim == 2:
        out = out[:, None, :]
    return out


@functools.partial(jax.jit, static_argnames=('interpret',))
def _run(R, Z, N, AM, NM, params, interpret=False):
    B = R.shape[0]
    L = params['layers']
    stacked = [
        _stack(L, 'phi_rbf', 'W'), _stack(L, 'phi_rbf', 'b'),
        _stack(L, 'phi_a', 0, 'W'), _stack(L, 'phi_a', 0, 'b'),
        _stack(L, 'phi_a', 1, 'W'), _stack(L, 'phi_a', 1, 'b'),
        _stack(L, 'phi_f', 'W'),
        _stack(L, 'phi_f_scale', 0, 'W'), _stack(L, 'phi_f_scale', 0, 'b'),
        _stack(L, 'phi_f_scale', 1, 'W'), _stack(L, 'phi_f_scale', 1, 'b'),
        _stack(L, 'phi_r', 0, 'W'), _stack(L, 'phi_r', 0, 'b'),
        _stack(L, 'phi_r', 1, 'W'), _stack(L, 'phi_r', 1, 'b'),
        _stack(L, 'phi_r_ext', 0, 'W'), _stack(L, 'phi_r_ext', 1, 'W'),
        _stack(L, 'phi_e', 0, 'W'), _stack(L, 'phi_e', 0, 'b'),
        _stack(L, 'phi_e', 1, 'W'), _stack(L, 'phi_e', 1, 'b'),
    ]
    atom = params['atomic']
    singles = [
        atom[0]['W'], atom[0]['b'].reshape(1, -1),
        atom[1]['W'], atom[1]['b'].reshape(1, -1),
        atom[2]['W'], atom[2]['b'].reshape(1, -1),
        params['emb'],
    ]
    weights = stacked + singles

    w_specs = [pl.BlockSpec(x.shape, lambda b, sh=x.shape: (0,) * len(sh))
               for x in weights]
    out_specs = [
        pl.BlockSpec((1, 1, 128), lambda b: (b, 0, 0)),
        pl.BlockSpec((1, A, 3), lambda b: (b, 0, 0)),
        pl.BlockSpec((1, A, 3), lambda b: (b, 0, 0)),
    ]
    out_shape = [
        jax.ShapeDtypeStruct((B, 1, 128), jnp.float32),
        jax.ShapeDtypeStruct((B, A, 3), jnp.float32),
        jax.ShapeDtypeStruct((B, A, 3), jnp.float32),
    ]
    vm = pltpu.VMEM
    scratch = [
        vm((E, 3), jnp.float32),
        vm((E, 3), jnp.float32),
        vm((E, 1), jnp.float32),
        vm((E, 1), jnp.float32),
        vm((E, RES), jnp.float32),
        vm((E, RES), jnp.float32),
        vm((E, 1), jnp.float32),
        vm((NI + 1, A, F), jnp.float32),
    ] + [vm((NI + 1, A, F), jnp.float32) for _ in range(6)] \
      + [vm((NI, A, F), jnp.float32) for _ in range(3)] \
      + [vm((A, F), jnp.float32),
         vm((A, F), jnp.float32),
         vm((3, A, F), jnp.float32),
         vm((E, 3), jnp.float32),
         vm((E, 1), jnp.float32),
         vm((E, RES), jnp.float32),
         vm((A, F), jnp.float32)]
    scratch += [vm((A, F), jnp.float32) for _ in range(6)]
    scratch += [vm((A, F), jnp.float32),
                vm((A, F), jnp.float32)]
    scratch += [vm((A, F), jnp.float32) for _ in range(3)]
    scratch += [vm((A, 3), jnp.float32),
                vm((A, 3), jnp.float32),
                vm((NI, A, 3), jnp.float32)]

    e3, ff, fdir = pl.pallas_call(
        _newton_kernel,
        grid=(B,),
        in_specs=[
            pl.BlockSpec((1, A, 3), lambda b: (b, 0, 0)),
            pl.BlockSpec((1, A, 1), lambda b: (b, 0, 0)),
            pl.BlockSpec((1, A, NN), lambda b: (b, 0, 0)),
            pl.BlockSpec((1, A, 1), lambda b: (b, 0, 0)),
            pl.BlockSpec((1, A, NN), lambda b: (b, 0, 0)),
        ] + w_specs,
        out_specs=out_specs,
        out_shape=out_shape,
        scratch_shapes=scratch,
        compiler_params=pltpu.CompilerParams(
            dimension_semantics=('arbitrary',),
            vmem_limit_bytes=100 * 1024 * 1024,
        ),
        interpret=interpret,
    )(R, Z.astype(jnp.int32)[..., None], N.astype(jnp.int32),
      AM[..., None], NM, *weights)
    return e3[:, 0, :1], ff, fdir


def kernel(R, Z, N, AM, NM, params):
    return _run(R, Z, N, AM, NM, params)


# speedup vs baseline: 1773.5983x; 1.0314x over previous
import functools

import jax
import jax.numpy as jnp
import numpy as np
from jax.experimental import pallas as pl
from jax.experimental.pallas import tpu as pltpu

A, NN, F, RES, NI = 128, 48, 128, 20, 3
E = A * NN
CUTOFF = 5.0
P = 9.0
EPS = 1e-8

C = 32
CE = C * NN
NC = A // C

C_A = (P + 1.0) * (P + 2.0) / 2.0
C_B = P * (P + 2.0)
C_C = P * (P + 1.0) / 2.0
C0 = float(np.sqrt(2.0 / CUTOFF))


PREC = jax.lax.Precision.HIGHEST


def _bf(x):
    return x.astype(jnp.bfloat16).astype(jnp.float32)


def _mm(x, w):
    return jax.lax.dot_general(x, w, (((1,), (1,)), ((), ())),
                               preferred_element_type=jnp.float32)


def _mmT(x, w):
    return jax.lax.dot_general(x, w, (((1,), (0,)), ((), ())),
                               preferred_element_type=jnp.float32)


def _swish(z):
    return z * jax.nn.sigmoid(z)


def _swish_d(z):
    s = jax.nn.sigmoid(z)
    return s * (1.0 + z * (1.0 - s))


def _rep(x):
    f = x.shape[-1]
    return jnp.broadcast_to(x[:, None, :], (C, NN, f)).reshape(CE, f)


def _seg(x):
    f = x.shape[-1]
    return jnp.sum(x.reshape(C, NN, f), axis=1)


def _mul_nm(x, nm):
    f = x.shape[-1]
    return (x.reshape(C, NN, f) * nm[:, :, None]).reshape(CE, f)


def _rowsum(x):
    return jnp.sum(x, axis=1, keepdims=True)


def _newton_kernel(
    r_ref, z_ref, n_ref, am_ref, nm_ref,
    wrbf, brbf, wa0, ba0, wa1, ba1, wf, wfs0, bfs0, wfs1, bfs1,
    wr0, br0, wr1, br1, wre0, wre1, we0, be0, we1, be1,
    w1, b1, w2, b2, w3, b3, emb,
    e_ref, ff_ref, fdir_ref,
    vec3_s, v3_s, d_s, rdp_s, cos_s, rbf_s, cut_s,
    aS, rS0, rS1, rS2, fS0, fS1, fS2, FiS0, FiS1, FiS2,
    am_s, pr_s, gFi_s,
    gV3_s, gcut_s, grbf_s, gA_s, gR0, gR1, gR2, gF0, gF1, gF2,
    gamS, gamG, grin0, grin1, grin2, ffS, ffG, fdir_acc, oh_s,
):
    rS = (rS0, rS1, rS2)
    fS = (fS0, fS1, fS2)
    FiS = (FiS0, FiS1, FiS2)
    gR = (gR0, gR1, gR2)
    gF = (gF0, gF1, gF2)
    grin = (grin0, grin1, grin2)

    Rm = r_ref[0]
    Zc = z_ref[0]
    AMc = am_ref[0]

    nvw = ((jax.lax.broadcasted_iota(jnp.int32, (1, RES), 1)
            .astype(jnp.float32) + 1.0) * (np.pi / CUTOFF))

    def build_onehot(k):
        Nk = n_ref[0, pl.ds(k * C, C), :]
        ids = jax.lax.broadcasted_iota(jnp.int32, (C, NN, A), 2)
        return (Nk[:, :, None] == ids).astype(jnp.bfloat16).reshape(CE, A)

    def chunk_onehot(k):
        return oh_s[pl.ds(k * CE, CE), :]

    def _split3(x):
        h = x.astype(jnp.bfloat16)
        r = x - h.astype(jnp.float32)
        m = r.astype(jnp.bfloat16)
        lo = (r - m.astype(jnp.float32)).astype(jnp.bfloat16)
        return h, m, lo

    def _dotn(a, b, dims):
        return jax.lax.dot_general(a, b, dims,
                                   preferred_element_type=jnp.float32)

    GAT_D = (((1,), (0,)), ((), ()))
    SCAT_D = (((0,), (0,)), ((), ()))

    def gat(ok, x):
        h, m, lo = _split3(x)
        return (_dotn(ok, h, GAT_D) + _dotn(ok, m, GAT_D)
                + _dotn(ok, lo, GAT_D))

    def scat(ok, y):
        h, m, lo = _split3(y)
        return (_dotn(ok, h, SCAT_D) + _dotn(ok, m, SCAT_D)
                + _dotn(ok, lo, SCAT_D))

    def nm_rows(k):
        return nm_ref[0, pl.ds(k * C, C), :]

    def geom_body(k, _):
        ok = build_onehot(k)
        oh_s[pl.ds(k * CE, CE), :] = ok
        Rg = gat(ok, Rm)
        Rk = r_ref[0, pl.ds(k * C, C), :]
        vec3 = Rg - _rep(Rk)
        d2 = _rowsum(vec3 * vec3) + EPS
        d = jnp.sqrt(d2)
        rdp = 1.0 / (d + EPS)
        sl = pl.ds(k * CE, CE)
        vec3_s[sl, :] = vec3
        v3_s[sl, :] = vec3 * rdp
        d_s[sl, :] = d
        rdp_s[sl, :] = rdp
        cos_s[sl, :] = jnp.cos(nvw * d)
        rbf_s[sl, :] = (C0 * jnp.sin(nvw * d)) * rdp
        xx = d / CUTOFF
        x4 = (xx * xx) * (xx * xx)
        x9 = x4 * x4 * xx
        cut_s[sl, :] = jnp.where(
            xx < 1.0,
            1.0 - C_A * x9 + C_B * (x9 * xx) - C_C * (x9 * xx * xx),
            0.0)
        return 0

    jax.lax.fori_loop(0, NC, geom_body, 0)

    zid = jax.lax.broadcasted_iota(jnp.int32, (A, 10), 1)
    a0 = jax.lax.dot_general((Zc == zid).astype(jnp.float32), emb[...],
                             (((1,), (0,)), ((), ())),
                             precision=PREC,
                             preferred_element_type=jnp.float32)
    aS[0] = a0
    zero_af = jnp.zeros((A, F), jnp.float32)
    for c in range(3):
        rS[c][0] = zero_af
        fS[c][0] = zero_af

    def fwd_layer(l, _):
        a_in = aS[l]
        za0 = _mm(a_in, wa0[l]) + ba0[l]
        am_s[...] = _mm(_swish(za0), wa1[l]) + ba1[l]
        zr0 = _mm(a_in, wr0[l]) + br0[l]
        pr_s[...] = _mm(_swish(zr0), wr1[l]) + br1[l]

        def body(k, _):
            ok = chunk_onehot(k)
            nmk = nm_rows(k)
            sl = pl.ds(k * CE, CE)
            sa = pl.ds(k * C, C)
            rbf_lin = _mm(rbf_s[sl, :], wrbf[l]) + brbf[l]
            rbf_m = rbf_lin * cut_s[sl, :]
            a_rep = _rep(am_s[sa, :])
            ag = gat(ok, am_s[...])
            msij = a_rep * ag * rbf_m
            s = _rowsum(_bf(msij) * _bf(wf[l]))
            v3 = v3_s[sl, :]
            fij3 = s * v3
            fdir_acc[l, sa, :] = jnp.sum(
                fij3.reshape(C, NN, 3) * nmk[:, :, None], axis=1)
            zfs0 = _mm(msij, wfs0[l]) + bfs0[l]
            fs = _mm(_swish(zfs0), wfs1[l]) + bfs1[l]
            zre0 = _mm(msij, wre0[l])
            pre = _mm(_swish(zre0), wre1[l])
            prk = pr_s[sa, :]
            for c in range(3):
                fi_kc = _seg(_mul_nm(fs * fij3[:, c:c + 1], nmk))
                FiS[c][l, sa, :] = fi_kc
                rg_kc = gat(ok, rS[c][l])
                drext = _seg(_mul_nm(pre * rg_kc, nmk))
                rS[c][l + 1, sa, :] = rS[c][l, sa, :] + prk * fi_kc + drext
                fS[c][l + 1, sa, :] = fS[c][l, sa, :] + fi_kc
            return 0

        jax.lax.fori_loop(0, NC, body, 0)

        de_raw = -(fS[0][l + 1] * rS[0][l + 1]
                   + fS[1][l + 1] * rS[1][l + 1]
                   + fS[2][l + 1] * rS[2][l + 1])
        ze0 = _mm(a_in, we0[l]) + be0[l]
        ee = _mm(_swish(ze0), we1[l]) + be1[l]
        aS[l + 1] = a_in + ee * de_raw
        return 0

    jax.lax.fori_loop(0, NI, fwd_layer, 0)

    a_fin = aS[NI]
    zh1 = _mm(a_fin, w1[...]) + b1[...]
    h1 = _swish(zh1)
    zh2 = _mm(h1, w2[...]) + b2[...]
    h2 = _swish(zh2)
    Ei = (_rowsum(_bf(h2) * _bf(w3[...])) + b3[0, 0]) * AMc
    Etot = jnp.sum(Ei)

    g_h2 = AMc * w3[...]
    g_h1 = _mmT(g_h2 * _swish_d(zh2), w2[...])
    gA_s[...] = _mmT(g_h1 * _swish_d(zh1), w1[...])
    for c in range(3):
        gR[c][...] = zero_af
        gF[c][...] = zero_af
    gV3_s[...] = jnp.zeros((E, 3), jnp.float32)
    gcut_s[...] = jnp.zeros((E, 1), jnp.float32)
    grbf_s[...] = jnp.zeros((E, RES), jnp.float32)

    def bwd_layer(i, _):
        l = NI - 1 - i
        a_in = aS[l]
        za0 = _mm(a_in, wa0[l]) + ba0[l]
        am_s[...] = _mm(_swish(za0), wa1[l]) + ba1[l]
        zr0 = _mm(a_in, wr0[l]) + br0[l]
        pr_s[...] = _mm(_swish(zr0), wr1[l]) + br1[l]
        ze0 = _mm(a_in, we0[l]) + be0[l]
        ee = _mm(_swish(ze0), we1[l]) + be1[l]

        de_raw = -(fS[0][l + 1] * rS[0][l + 1]
                   + fS[1][l + 1] * rS[1][l + 1]
                   + fS[2][l + 1] * rS[2][l + 1])
        g_a = gA_s[...]
        g_ee = g_a * de_raw
        g_deraw = g_a * ee
        for c in range(3):
            gF[c][...] = gF[c][...] - g_deraw * rS[c][l + 1]
            gR[c][...] = gR[c][...] - g_deraw * fS[c][l + 1]
        g_pr = (gR[0][...] * FiS[0][l] + gR[1][...] * FiS[1][l]
                + gR[2][...] * FiS[2][l])
        prv = pr_s[...]
        for c in range(3):
            gFi_s[c] = gR[c][...] * prv + gF[c][...]
            grin[c][...] = zero_af
        gamS[...] = zero_af

        def body(k, _):
            ok = chunk_onehot(k)
            nmk = nm_rows(k)
            sl = pl.ds(k * CE, CE)
            sa = pl.ds(k * C, C)
            rbf_lin = _mm(rbf_s[sl, :], wrbf[l]) + brbf[l]
            cutk = cut_s[sl, :]
            rbf_m = rbf_lin * cutk
            a_rep = _rep(am_s[sa, :])
            ag = gat(ok, am_s[...])
            msij = a_rep * ag * rbf_m
            s = _rowsum(_bf(msij) * _bf(wf[l]))
            v3 = v3_s[sl, :]
            fij3 = s * v3
            zfs0 = _mm(msij, wfs0[l]) + bfs0[l]
            fs = _mm(_swish(zfs0), wfs1[l]) + bfs1[l]
            zre0 = _mm(msij, wre0[l])
            pre = _mm(_swish(zre0), wre1[l])

            g_fs = jnp.zeros((CE, F), jnp.float32)
            g_s = jnp.zeros((CE, 1), jnp.float32)
            gv3_cols = []
            for c in range(3):
                g_fij2 = _mul_nm(_rep(gFi_s[c, pl.ds(k * C, C), :]), nmk)
                g_fs = g_fs + g_fij2 * fij3[:, c:c + 1]
                g_fij_c = _rowsum(g_fij2 * fs)
                g_s = g_s + g_fij_c * v3[:, c:c + 1]
                gv3_cols.append(g_fij_c * s)
            gV3_s[sl, :] = gV3_s[sl, :] + jnp.concatenate(gv3_cols, axis=1)

            g_pre = jnp.zeros((CE, F), jnp.float32)
            for c in range(3):
                rg_kc = gat(ok, rS[c][l])
                gdx = _mul_nm(_rep(gR[c][sa, :]), nmk)
                g_pre = g_pre + gdx * rg_kc
                grin[c][...] = grin[c][...] + scat(ok, gdx * pre)

            g_msij = _mmT(_mmT(g_pre, wre1[l]) * _swish_d(zre0), wre0[l])
            g_msij = g_msij + _mmT(
                _mmT(g_fs, wfs1[l]) * _swish_d(zfs0), wfs0[l])
            g_msij = g_msij + g_s * wf[l]

            g_arep = g_msij * ag * rbf_m
            g_ag = g_msij * a_rep * rbf_m
            g_rbfm = g_msij * a_rep * ag
            gamG[sa, :] = _seg(g_arep)
            gamS[...] = gamS[...] + scat(ok, g_ag)
            gcut_s[sl, :] = gcut_s[sl, :] + _rowsum(g_rbfm * rbf_lin)
            grbf_s[sl, :] = grbf_s[sl, :] + _mmT(g_rbfm * cutk, wrbf[l])
            return 0

        jax.lax.fori_loop(0, NC, body, 0)

        g_am = gamG[...] + gamS[...]
        g_a_in = g_a + _mmT(_mmT(g_ee, we1[l]) * _swish_d(ze0), we0[l])
        g_a_in = g_a_in + _mmT(_mmT(g_pr, wr1[l]) * _swish_d(zr0), wr0[l])
        g_a_in = g_a_in + _mmT(_mmT(g_am, wa1[l]) * _swish_d(za0), wa0[l])
        gA_s[...] = g_a_in
        for c in range(3):
            gR[c][...] = gR[c][...] + grin[c][...]
        return 0

    jax.lax.fori_loop(0, NI, bwd_layer, 0)

    ffS[...] = jnp.zeros((A, 3), jnp.float32)

    def geom_bwd(k, _):
        ok = chunk_onehot(k)
        sl = pl.ds(k * CE, CE)
        sa = pl.ds(k * C, C)
        d = d_s[sl, :]
        rdp = rdp_s[sl, :]
        vec3 = vec3_s[sl, :]
        grbf = grbf_s[sl, :]
        gv3 = gV3_s[sl, :]
        g_d = _rowsum(grbf * ((C0 * nvw) * cos_s[sl, :])) * rdp
        g_dp = -_rowsum(grbf * rbf_s[sl, :]) * rdp
        xx = d / CUTOFF
        x4 = (xx * xx) * (xx * xx)
        x8 = x4 * x4
        dcut = jnp.where(
            xx < 1.0,
            (-9.0 * C_A) * x8 + (10.0 * C_B) * (x8 * xx)
            - (11.0 * C_C) * (x8 * xx * xx),
            0.0) / CUTOFF
        g_d = g_d + gcut_s[sl, :] * dcut
        g_vec3 = gv3 * rdp
        g_dp = g_dp - _rowsum(gv3 * vec3) * (rdp * rdp)
        g_d = g_d + g_dp
        g_vec3 = g_vec3 + (g_d / d) * vec3
        ffS[...] = ffS[...] + scat(ok, g_vec3)
        ffG[sa, :] = _seg(g_vec3)
        return 0

    jax.lax.fori_loop(0, NC, geom_bwd, 0)

    e_ref[...] = jnp.broadcast_to(Etot, (1, 1, 128))
    ff_ref[...] = (ffG[...] - ffS[...])[None]
    fdir_ref[...] = (fdir_acc[0] + fdir_acc[1] + fdir_acc[2])[None]


def _stack(layers, *path):
    def get(lp):
        v = lp
        for p in path:
            v = v[p]
        return v
    out = jnp.stack([get(lp) for lp in layers])
    if out.ndim == 2:
        out = out[:, None, :]
    return out


@functools.partial(jax.jit, static_argnames=('interpret',))
def _run(R, Z, N, AM, NM, params, interpret=False):
    B = R.shape[0]
    L = params['layers']
    stacked = [
        _stack(L, 'phi_rbf', 'W'), _stack(L, 'phi_rbf', 'b'),
        _stack(L, 'phi_a', 0, 'W'), _stack(L, 'phi_a', 0, 'b'),
        _stack(L, 'phi_a', 1, 'W'), _stack(L, 'phi_a', 1, 'b'),
        _stack(L, 'phi_f', 'W'),
        _stack(L, 'phi_f_scale', 0, 'W'), _stack(L, 'phi_f_scale', 0, 'b'),
        _stack(L, 'phi_f_scale', 1, 'W'), _stack(L, 'phi_f_scale', 1, 'b'),
        _stack(L, 'phi_r', 0, 'W'), _stack(L, 'phi_r', 0, 'b'),
        _stack(L, 'phi_r', 1, 'W'), _stack(L, 'phi_r', 1, 'b'),
        _stack(L, 'phi_r_ext', 0, 'W'), _stack(L, 'phi_r_ext', 1, 'W'),
        _stack(L, 'phi_e', 0, 'W'), _stack(L, 'phi_e', 0, 'b'),
        _stack(L, 'phi_e', 1, 'W'), _stack(L, 'phi_e', 1, 'b'),
    ]
    atom = params['atomic']
    singles = [
        atom[0]['W'], atom[0]['b'].reshape(1, -1),
        atom[1]['W'], atom[1]['b'].reshape(1, -1),
        atom[2]['W'], atom[2]['b'].reshape(1, -1),
        params['emb'],
    ]
    weights = stacked + singles

    w_specs = [pl.BlockSpec(x.shape, lambda b, sh=x.shape: (0,) * len(sh))
               for x in weights]
    out_specs = [
        pl.BlockSpec((1, 1, 128), lambda b: (b, 0, 0)),
        pl.BlockSpec((1, A, 3), lambda b: (b, 0, 0)),
        pl.BlockSpec((1, A, 3), lambda b: (b, 0, 0)),
    ]
    out_shape = [
        jax.ShapeDtypeStruct((B, 1, 128), jnp.float32),
        jax.ShapeDtypeStruct((B, A, 3), jnp.float32),
        jax.ShapeDtypeStruct((B, A, 3), jnp.float32),
    ]
    vm = pltpu.VMEM
    scratch = [
        vm((E, 3), jnp.float32),
        vm((E, 3), jnp.float32),
        vm((E, 1), jnp.float32),
        vm((E, 1), jnp.float32),
        vm((E, RES), jnp.float32),
        vm((E, RES), jnp.float32),
        vm((E, 1), jnp.float32),
        vm((NI + 1, A, F), jnp.float32),
    ] + [vm((NI + 1, A, F), jnp.float32) for _ in range(6)] \
      + [vm((NI, A, F), jnp.float32) for _ in range(3)] \
      + [vm((A, F), jnp.float32),
         vm((A, F), jnp.float32),
         vm((3, A, F), jnp.float32),
         vm((E, 3), jnp.float32),
         vm((E, 1), jnp.float32),
         vm((E, RES), jnp.float32),
         vm((A, F), jnp.float32)]
    scratch += [vm((A, F), jnp.float32) for _ in range(6)]
    scratch += [vm((A, F), jnp.float32),
                vm((A, F), jnp.float32)]
    scratch += [vm((A, F), jnp.float32) for _ in range(3)]
    scratch += [vm((A, 3), jnp.float32),
                vm((A, 3), jnp.float32),
                vm((NI, A, 3), jnp.float32),
                vm((E, A), jnp.bfloat16)]

    e3, ff, fdir = pl.pallas_call(
        _newton_kernel,
        grid=(B,),
        in_specs=[
            pl.BlockSpec((1, A, 3), lambda b: (b, 0, 0)),
            pl.BlockSpec((1, A, 1), lambda b: (b, 0, 0)),
            pl.BlockSpec((1, A, NN), lambda b: (b, 0, 0)),
            pl.BlockSpec((1, A, 1), lambda b: (b, 0, 0)),
            pl.BlockSpec((1, A, NN), lambda b: (b, 0, 0)),
        ] + w_specs,
        out_specs=out_specs,
        out_shape=out_shape,
        scratch_shapes=scratch,
        compiler_params=pltpu.CompilerParams(
            dimension_semantics=('arbitrary',),
            vmem_limit_bytes=100 * 1024 * 1024,
        ),
        interpret=interpret,
    )(R, Z.astype(jnp.int32)[..., None], N.astype(jnp.int32),
      AM[..., None], NM, *weights)
    return e3[:, 0, :1], ff, fdir


def kernel(R, Z, N, AM, NM, params):
    return _run(R, Z, N, AM, NM, params)
